# baseline (speedup 1.0000x reference)
import jax
import jax.numpy as jnp
from jax import lax
from jax.experimental import pallas as pl
from jax.experimental.pallas import tpu as pltpu, tpu_sc as plsc

NUM_CORES = 2
NUM_SUBCORES = 16
NW = NUM_CORES * NUM_SUBCORES
LANES = 16
CHUNK = 128


def _make_sc_call(batch, dim):
    bpw = batch // NW
    nchunk = bpw // CHUNK

    def body(user_ids, item_ids, user_table, item_table, wb,
             out, uidx, iidx, urows, irows, wvec, logit, sem):
        wid = lax.axis_index("s") * NUM_CORES + lax.axis_index("c")
        base = wid * bpw

        pltpu.sync_copy(user_ids.at[pl.ds(base, bpw)], uidx)
        pltpu.sync_copy(item_ids.at[pl.ds(base, bpw)], iidx)
        pltpu.sync_copy(wb, wvec)

        copies = []
        for c in range(nchunk):
            sl = pl.ds(c * CHUNK, CHUNK)
            copies.append(pltpu.async_copy(
                user_table.at[uidx.at[sl]], urows.at[sl], sem))
            copies.append(pltpu.async_copy(
                item_table.at[iidx.at[sl]], irows.at[sl], sem))
        for cp in copies:
            cp.wait()

        wu0 = wvec[pl.ds(0, LANES)]
        wu1 = wvec[pl.ds(LANES, LANES)]
        wi0 = wvec[pl.ds(2 * LANES, LANES)]
        wi1 = wvec[pl.ds(3 * LANES, LANES)]
        bv = wvec[pl.ds(4 * LANES, LANES)]

        last = lax.iota(jnp.int32, LANES) == (LANES - 1)

        def elem(i, _):
            acc = (urows[i, pl.ds(0, LANES)] * wu0
                   + urows[i, pl.ds(LANES, LANES)] * wu1
                   + irows[i, pl.ds(0, LANES)] * wi0
                   + irows[i, pl.ds(LANES, LANES)] * wi1)
            total = plsc.cumsum(acc)
            idx = jnp.full((LANES,), i, jnp.int32)
            plsc.store_scatter(logit, [idx], total, mask=last)
            return _

        lax.fori_loop(0, bpw, elem, None)

        for g in range(bpw // LANES):
            sl = pl.ds(g * LANES, LANES)
            v = logit[sl] + bv
            logit[sl] = 1.0 / (1.0 + jnp.exp(-v))

        pltpu.sync_copy(logit, out.at[pl.ds(base, bpw)])

    return pl.kernel(
        body,
        out_type=jax.ShapeDtypeStruct((batch,), jnp.float32),
        mesh=plsc.VectorSubcoreMesh(
            core_axis_name="c", subcore_axis_name="s",
            num_cores=NUM_CORES, num_subcores=NUM_SUBCORES),
        compiler_params=pltpu.CompilerParams(
            needs_layout_passes=False, use_tc_tiling_on_sc=False),
        scratch_types=[
            pltpu.VMEM((bpw,), jnp.int32),
            pltpu.VMEM((bpw,), jnp.int32),
            pltpu.VMEM((bpw, dim), jnp.float32),
            pltpu.VMEM((bpw, dim), jnp.float32),
            pltpu.VMEM((5 * LANES,), jnp.float32),
            pltpu.VMEM((bpw,), jnp.float32),
            pltpu.SemaphoreType.DMA,
        ],
    )


@jax.jit
def kernel(user_ids, item_ids, user_table, item_table, W, b):
    batch = user_ids.shape[0]
    dim = user_table.shape[1]
    wb = jnp.concatenate(
        [W.reshape(-1), jnp.full((LANES,), b[0], jnp.float32)])
    call = _make_sc_call(batch, dim)
    out = call(user_ids.astype(jnp.int32), item_ids.astype(jnp.int32),
               user_table, item_table, wb)
    return out.reshape(batch, 1)


# speedup vs baseline: 2.7219x; 2.7219x over previous
import functools

import jax
import jax.numpy as jnp
from jax import lax
from jax.experimental import pallas as pl
from jax.experimental.pallas import tpu as pltpu, tpu_sc as plsc

NUM_CORES = 2
NUM_SUBCORES = 16
NW = NUM_CORES * NUM_SUBCORES
LANES = 16
CHUNK = 128
BLK = 2048


def _tc_body(dim, wb_ref, ut_ref, it_ref, pu_ref, pi_ref):
    acc_u = jnp.zeros((BLK,), jnp.float32) + wb_ref[2 * dim]
    acc_i = jnp.zeros((BLK,), jnp.float32)
    for j in range(dim):
        acc_u += ut_ref[j, :] * wb_ref[j]
        acc_i += it_ref[j, :] * wb_ref[dim + j]
    pu_ref[:] = acc_u
    pi_ref[:] = acc_i


def _make_tc_call(n_rows, dim):
    grid = (pl.cdiv(n_rows, BLK),)
    return pl.pallas_call(
        functools.partial(_tc_body, dim),
        grid=grid,
        in_specs=[
            pl.BlockSpec(memory_space=pltpu.SMEM),
            pl.BlockSpec((dim, BLK), lambda g: (0, g)),
            pl.BlockSpec((dim, BLK), lambda g: (0, g)),
        ],
        out_specs=[
            pl.BlockSpec((BLK,), lambda g: (g,)),
            pl.BlockSpec((BLK,), lambda g: (g,)),
        ],
        out_shape=[
            jax.ShapeDtypeStruct((n_rows,), jnp.float32),
            jax.ShapeDtypeStruct((n_rows,), jnp.float32),
        ],
    )


def _sc_body(bpw, user_ids, item_ids, pu, pi, out,
             uidx, iidx, gu, gi, sem):
    wid = lax.axis_index("s") * NUM_CORES + lax.axis_index("c")
    base = wid * bpw

    pltpu.sync_copy(user_ids.at[pl.ds(base, bpw)], uidx)
    pltpu.sync_copy(item_ids.at[pl.ds(base, bpw)], iidx)

    copies = []
    for c in range(bpw // CHUNK):
        sl = pl.ds(c * CHUNK, CHUNK)
        copies.append(pltpu.async_copy(pu.at[uidx.at[sl]], gu.at[sl], sem))
        copies.append(pltpu.async_copy(pi.at[iidx.at[sl]], gi.at[sl], sem))
    for cp in copies:
        cp.wait()

    for g in range(bpw // LANES):
        sl = pl.ds(g * LANES, LANES)
        s = gu[sl] + gi[sl]
        gu[sl] = 1.0 / (1.0 + jnp.exp(-s))

    pltpu.sync_copy(gu, out.at[pl.ds(base, bpw)])


def _make_sc_call(batch):
    bpw = batch // NW
    return pl.kernel(
        functools.partial(_sc_body, bpw),
        out_type=jax.ShapeDtypeStruct((batch,), jnp.float32),
        mesh=plsc.VectorSubcoreMesh(
            core_axis_name="c", subcore_axis_name="s",
            num_cores=NUM_CORES, num_subcores=NUM_SUBCORES),
        compiler_params=pltpu.CompilerParams(
            needs_layout_passes=False, use_tc_tiling_on_sc=False),
        scratch_types=[
            pltpu.VMEM((bpw,), jnp.int32),
            pltpu.VMEM((bpw,), jnp.int32),
            pltpu.VMEM((bpw,), jnp.float32),
            pltpu.VMEM((bpw,), jnp.float32),
            pltpu.SemaphoreType.DMA,
        ],
    )


@jax.jit
def kernel(user_ids, item_ids, user_table, item_table, W, b):
    batch = user_ids.shape[0]
    n_rows, dim = user_table.shape
    wb = jnp.concatenate(
        [W.reshape(-1), jnp.full((LANES,), b[0], jnp.float32)])
    pu, pi = _make_tc_call(n_rows, dim)(wb, user_table.T, item_table.T)
    out = _make_sc_call(batch)(
        user_ids.astype(jnp.int32), item_ids.astype(jnp.int32), pu, pi)
    return out.reshape(batch, 1)


# speedup vs baseline: 5.6320x; 2.0691x over previous
import functools

import jax
import jax.numpy as jnp
from jax import lax
from jax.experimental import pallas as pl
from jax.experimental.pallas import tpu as pltpu, tpu_sc as plsc

NUM_CORES = 2
NUM_SUBCORES = 16
NW = NUM_CORES * NUM_SUBCORES
LANES = 16
CHUNK = 128
BLK = 8192


def _tc_body(dim, wb_ref, wmat_ref, ut_ref, it_ref, pu_ref, pi_ref):
    acc_u = ut_ref[pl.ds(0, 8), :] * wmat_ref[pl.ds(0, 8), 0:1]
    acc_i = it_ref[pl.ds(0, 8), :] * wmat_ref[pl.ds(0, 8), 1:2]
    for k in range(1, dim // 8):
        sl = pl.ds(8 * k, 8)
        acc_u += ut_ref[sl, :] * wmat_ref[sl, 0:1]
        acc_i += it_ref[sl, :] * wmat_ref[sl, 1:2]
    pu_ref[:] = jnp.sum(acc_u, axis=0) + wb_ref[2 * dim]
    pi_ref[:] = jnp.sum(acc_i, axis=0)


def _make_tc_call(n_rows, dim):
    grid = (pl.cdiv(n_rows, BLK),)
    return pl.pallas_call(
        functools.partial(_tc_body, dim),
        grid=grid,
        in_specs=[
            pl.BlockSpec(memory_space=pltpu.SMEM),
            pl.BlockSpec((dim, 2), lambda g: (0, 0)),
            pl.BlockSpec((dim, BLK), lambda g: (0, g)),
            pl.BlockSpec((dim, BLK), lambda g: (0, g)),
        ],
        out_specs=[
            pl.BlockSpec((BLK,), lambda g: (g,)),
            pl.BlockSpec((BLK,), lambda g: (g,)),
        ],
        out_shape=[
            jax.ShapeDtypeStruct((n_rows,), jnp.float32),
            jax.ShapeDtypeStruct((n_rows,), jnp.float32),
        ],
    )


def _sc_body(bpw, user_ids, item_ids, pu, pi, out,
             uidx, iidx, gu, gi, sem):
    wid = lax.axis_index("s") * NUM_CORES + lax.axis_index("c")
    base = wid * bpw

    pltpu.sync_copy(user_ids.at[pl.ds(base, bpw)], uidx)
    pltpu.sync_copy(item_ids.at[pl.ds(base, bpw)], iidx)

    copies = []
    for c in range(bpw // CHUNK):
        sl = pl.ds(c * CHUNK, CHUNK)
        copies.append(pltpu.async_copy(pu.at[uidx.at[sl]], gu.at[sl], sem))
        copies.append(pltpu.async_copy(pi.at[iidx.at[sl]], gi.at[sl], sem))
    for cp in copies:
        cp.wait()

    for g in range(bpw // LANES):
        sl = pl.ds(g * LANES, LANES)
        s = gu[sl] + gi[sl]
        gu[sl] = 1.0 / (1.0 + jnp.exp(-s))

    pltpu.sync_copy(gu, out.at[pl.ds(base, bpw)])


def _make_sc_call(batch):
    bpw = batch // NW
    return pl.kernel(
        functools.partial(_sc_body, bpw),
        out_type=jax.ShapeDtypeStruct((batch,), jnp.float32),
        mesh=plsc.VectorSubcoreMesh(
            core_axis_name="c", subcore_axis_name="s",
            num_cores=NUM_CORES, num_subcores=NUM_SUBCORES),
        compiler_params=pltpu.CompilerParams(
            needs_layout_passes=False, use_tc_tiling_on_sc=False),
        scratch_types=[
            pltpu.VMEM((bpw,), jnp.int32),
            pltpu.VMEM((bpw,), jnp.int32),
            pltpu.VMEM((bpw,), jnp.float32),
            pltpu.VMEM((bpw,), jnp.float32),
            pltpu.SemaphoreType.DMA,
        ],
    )


@jax.jit
def kernel(user_ids, item_ids, user_table, item_table, W, b):
    batch = user_ids.shape[0]
    n_rows, dim = user_table.shape
    wb = jnp.concatenate(
        [W.reshape(-1), jnp.full((LANES,), b[0], jnp.float32)])
    wmat = W.reshape(2, dim).T
    pu, pi = _make_tc_call(n_rows, dim)(
        wb, wmat, user_table.T, item_table.T)
    out = _make_sc_call(batch)(
        user_ids.astype(jnp.int32), item_ids.astype(jnp.int32), pu, pi)
    return out.reshape(batch, 1)


# speedup vs baseline: 6.0071x; 1.0666x over previous
import functools

import jax
import jax.numpy as jnp
from jax import lax
from jax.experimental import pallas as pl
from jax.experimental.pallas import tpu as pltpu, tpu_sc as plsc

NUM_CORES = 2
NUM_SUBCORES = 16
NW = NUM_CORES * NUM_SUBCORES
LANES = 16
CHUNK = 128
BLK = 8192


def _tc_body(dim, wb_ref, wrows_ref, ut_ref, it_ref, pu_ref, pi_ref):
    dn = (((1,), (0,)), ((), ()))
    ru = jax.lax.dot_general(wrows_ref[0:1, :], ut_ref[...], dn,
                             preferred_element_type=jnp.float32)
    ri = jax.lax.dot_general(wrows_ref[1:2, :], it_ref[...], dn,
                             preferred_element_type=jnp.float32)
    pu_ref[:] = ru.reshape(ru.shape[1]) + wb_ref[2 * dim]
    pi_ref[:] = ri.reshape(ri.shape[1])


def _make_tc_call(n_rows, dim):
    grid = (pl.cdiv(n_rows, BLK),)
    return pl.pallas_call(
        functools.partial(_tc_body, dim),
        grid=grid,
        in_specs=[
            pl.BlockSpec(memory_space=pltpu.SMEM),
            pl.BlockSpec((2, dim), lambda g: (0, 0)),
            pl.BlockSpec((dim, BLK), lambda g: (0, g)),
            pl.BlockSpec((dim, BLK), lambda g: (0, g)),
        ],
        out_specs=[
            pl.BlockSpec((BLK,), lambda g: (g,)),
            pl.BlockSpec((BLK,), lambda g: (g,)),
        ],
        out_shape=[
            jax.ShapeDtypeStruct((n_rows,), jnp.float32),
            jax.ShapeDtypeStruct((n_rows,), jnp.float32),
        ],
    )


def _sc_body(bpw, user_ids, item_ids, pu, pi, out,
             uidx, iidx, gu, gi, sem):
    wid = lax.axis_index("s") * NUM_CORES + lax.axis_index("c")
    base = wid * bpw

    pltpu.sync_copy(user_ids.at[pl.ds(base, bpw)], uidx)
    pltpu.sync_copy(item_ids.at[pl.ds(base, bpw)], iidx)

    copies = []
    for c in range(bpw // CHUNK):
        sl = pl.ds(c * CHUNK, CHUNK)
        copies.append(pltpu.async_copy(pu.at[uidx.at[sl]], gu.at[sl], sem))
        copies.append(pltpu.async_copy(pi.at[iidx.at[sl]], gi.at[sl], sem))
    for cp in copies:
        cp.wait()

    for g in range(bpw // LANES):
        sl = pl.ds(g * LANES, LANES)
        s = gu[sl] + gi[sl]
        gu[sl] = 1.0 / (1.0 + jnp.exp(-s))

    pltpu.sync_copy(gu, out.at[pl.ds(base, bpw)])


def _make_sc_call(batch):
    bpw = batch // NW
    return pl.kernel(
        functools.partial(_sc_body, bpw),
        out_type=jax.ShapeDtypeStruct((batch,), jnp.float32),
        mesh=plsc.VectorSubcoreMesh(
            core_axis_name="c", subcore_axis_name="s",
            num_cores=NUM_CORES, num_subcores=NUM_SUBCORES),
        compiler_params=pltpu.CompilerParams(
            needs_layout_passes=False, use_tc_tiling_on_sc=False),
        scratch_types=[
            pltpu.VMEM((bpw,), jnp.int32),
            pltpu.VMEM((bpw,), jnp.int32),
            pltpu.VMEM((bpw,), jnp.float32),
            pltpu.VMEM((bpw,), jnp.float32),
            pltpu.SemaphoreType.DMA,
        ],
    )


@jax.jit
def kernel(user_ids, item_ids, user_table, item_table, W, b):
    batch = user_ids.shape[0]
    n_rows, dim = user_table.shape
    wb = jnp.concatenate(
        [W.reshape(-1), jnp.full((LANES,), b[0], jnp.float32)])
    wrows = W.reshape(2, dim)
    pu, pi = _make_tc_call(n_rows, dim)(
        wb, wrows, user_table.T, item_table.T)
    out = _make_sc_call(batch)(
        user_ids.astype(jnp.int32), item_ids.astype(jnp.int32), pu, pi)
    return out.reshape(batch, 1)


# speedup vs baseline: 7.9238x; 1.3191x over previous
import functools

import jax
import jax.numpy as jnp
from jax import lax
from jax.experimental import pallas as pl
from jax.experimental.pallas import tpu as pltpu, tpu_sc as plsc

NUM_CORES = 2
NUM_SUBCORES = 16
NW = NUM_CORES * NUM_SUBCORES
LANES = 16
CHUNK = 128
BLK = 16384


def _tc_body(dim, wb_ref, wrows_ref, ut_ref, it_ref, pu_ref, pi_ref):
    dn = (((1,), (0,)), ((), ()))
    ru = jax.lax.dot_general(wrows_ref[0:1, :], ut_ref[...], dn,
                             preferred_element_type=jnp.float32)
    ri = jax.lax.dot_general(wrows_ref[1:2, :], it_ref[...], dn,
                             preferred_element_type=jnp.float32)
    pu_ref[:] = ru.reshape(ru.shape[1]) + wb_ref[2 * dim]
    pi_ref[:] = ri.reshape(ri.shape[1])


def _make_tc_call(n_rows, dim):
    grid = (pl.cdiv(n_rows, BLK),)
    return pl.pallas_call(
        functools.partial(_tc_body, dim),
        grid=grid,
        in_specs=[
            pl.BlockSpec(memory_space=pltpu.SMEM),
            pl.BlockSpec((2, dim), lambda g: (0, 0)),
            pl.BlockSpec((dim, BLK), lambda g: (0, g)),
            pl.BlockSpec((dim, BLK), lambda g: (0, g)),
        ],
        out_specs=[
            pl.BlockSpec((BLK,), lambda g: (g,)),
            pl.BlockSpec((BLK,), lambda g: (g,)),
        ],
        out_shape=[
            jax.ShapeDtypeStruct((n_rows,), jnp.float32),
            jax.ShapeDtypeStruct((n_rows,), jnp.float32),
        ],
    )


def _sc_body(bpw, user_ids, item_ids, pu, pi, out,
             uidx, iidx, gu, gi, sem):
    wid = lax.axis_index("s") * NUM_CORES + lax.axis_index("c")
    base = wid * bpw

    pltpu.sync_copy(user_ids.at[pl.ds(base, bpw)], uidx)
    pltpu.sync_copy(item_ids.at[pl.ds(base, bpw)], iidx)

    copies = []
    for c in range(bpw // CHUNK):
        sl = pl.ds(c * CHUNK, CHUNK)
        copies.append(pltpu.async_copy(pu.at[uidx.at[sl]], gu.at[sl], sem))
        copies.append(pltpu.async_copy(pi.at[iidx.at[sl]], gi.at[sl], sem))
    for cp in copies:
        cp.wait()

    for g in range(bpw // LANES):
        sl = pl.ds(g * LANES, LANES)
        s = gu[sl] + gi[sl]
        gu[sl] = 1.0 / (1.0 + jnp.exp(-s))

    pltpu.sync_copy(gu, out.at[pl.ds(base, bpw)])


def _make_sc_call(batch):
    bpw = batch // NW
    return pl.kernel(
        functools.partial(_sc_body, bpw),
        out_type=jax.ShapeDtypeStruct((batch,), jnp.float32),
        mesh=plsc.VectorSubcoreMesh(
            core_axis_name="c", subcore_axis_name="s",
            num_cores=NUM_CORES, num_subcores=NUM_SUBCORES),
        compiler_params=pltpu.CompilerParams(
            needs_layout_passes=False, use_tc_tiling_on_sc=False),
        scratch_types=[
            pltpu.VMEM((bpw,), jnp.int32),
            pltpu.VMEM((bpw,), jnp.int32),
            pltpu.VMEM((bpw,), jnp.float32),
            pltpu.VMEM((bpw,), jnp.float32),
            pltpu.SemaphoreType.DMA,
        ],
    )


@jax.jit
def kernel(user_ids, item_ids, user_table, item_table, W, b):
    batch = user_ids.shape[0]
    n_rows, dim = user_table.shape
    wb = jnp.concatenate(
        [W.reshape(-1), jnp.full((LANES,), b[0], jnp.float32)])
    wrows = W.reshape(2, dim)
    pu, pi = _make_tc_call(n_rows, dim)(
        wb, wrows, user_table.T, item_table.T)
    out = _make_sc_call(batch)(
        user_ids.astype(jnp.int32), item_ids.astype(jnp.int32), pu, pi)
    return out.reshape(batch, 1)


# speedup vs baseline: 8.7006x; 1.0980x over previous
import functools

import jax
import jax.numpy as jnp
from jax import lax
from jax.experimental import pallas as pl
from jax.experimental.pallas import tpu as pltpu, tpu_sc as plsc

NUM_CORES = 2
NUM_SUBCORES = 16
NW = NUM_CORES * NUM_SUBCORES
LANES = 16
CHUNK = 128
BLK = 32768


def _tc_body(dim, wb_ref, wrows_ref, ut_ref, it_ref, pu_ref, pi_ref):
    dn = (((1,), (0,)), ((), ()))
    ru = jax.lax.dot_general(wrows_ref[0:1, :], ut_ref[...], dn,
                             preferred_element_type=jnp.float32)
    ri = jax.lax.dot_general(wrows_ref[1:2, :], it_ref[...], dn,
                             preferred_element_type=jnp.float32)
    pu_ref[:] = ru.reshape(ru.shape[1]) + wb_ref[2 * dim]
    pi_ref[:] = ri.reshape(ri.shape[1])


def _make_tc_call(n_rows, dim):
    grid = (pl.cdiv(n_rows, BLK),)
    return pl.pallas_call(
        functools.partial(_tc_body, dim),
        grid=grid,
        in_specs=[
            pl.BlockSpec(memory_space=pltpu.SMEM),
            pl.BlockSpec((2, dim), lambda g: (0, 0)),
            pl.BlockSpec((dim, BLK), lambda g: (0, g)),
            pl.BlockSpec((dim, BLK), lambda g: (0, g)),
        ],
        out_specs=[
            pl.BlockSpec((BLK,), lambda g: (g,)),
            pl.BlockSpec((BLK,), lambda g: (g,)),
        ],
        out_shape=[
            jax.ShapeDtypeStruct((n_rows,), jnp.float32),
            jax.ShapeDtypeStruct((n_rows,), jnp.float32),
        ],
    )


def _sc_body(bpw, user_ids, item_ids, pu, pi, out,
             uidx, iidx, gu, gi, sem):
    wid = lax.axis_index("s") * NUM_CORES + lax.axis_index("c")
    base = wid * bpw

    pltpu.sync_copy(user_ids.at[pl.ds(base, bpw)], uidx)
    pltpu.sync_copy(item_ids.at[pl.ds(base, bpw)], iidx)

    copies = []
    for c in range(bpw // CHUNK):
        sl = pl.ds(c * CHUNK, CHUNK)
        copies.append(pltpu.async_copy(pu.at[uidx.at[sl]], gu.at[sl], sem))
        copies.append(pltpu.async_copy(pi.at[iidx.at[sl]], gi.at[sl], sem))
    for cp in copies:
        cp.wait()

    for g in range(bpw // LANES):
        sl = pl.ds(g * LANES, LANES)
        s = gu[sl] + gi[sl]
        gu[sl] = 1.0 / (1.0 + jnp.exp(-s))

    pltpu.sync_copy(gu, out.at[pl.ds(base, bpw)])


def _make_sc_call(batch):
    bpw = batch // NW
    return pl.kernel(
        functools.partial(_sc_body, bpw),
        out_type=jax.ShapeDtypeStruct((batch,), jnp.float32),
        mesh=plsc.VectorSubcoreMesh(
            core_axis_name="c", subcore_axis_name="s",
            num_cores=NUM_CORES, num_subcores=NUM_SUBCORES),
        compiler_params=pltpu.CompilerParams(
            needs_layout_passes=False, use_tc_tiling_on_sc=False),
        scratch_types=[
            pltpu.VMEM((bpw,), jnp.int32),
            pltpu.VMEM((bpw,), jnp.int32),
            pltpu.VMEM((bpw,), jnp.float32),
            pltpu.VMEM((bpw,), jnp.float32),
            pltpu.SemaphoreType.DMA,
        ],
    )


@jax.jit
def kernel(user_ids, item_ids, user_table, item_table, W, b):
    batch = user_ids.shape[0]
    n_rows, dim = user_table.shape
    wb = jnp.concatenate(
        [W.reshape(-1), jnp.full((LANES,), b[0], jnp.float32)])
    wrows = W.reshape(2, dim)
    pu, pi = _make_tc_call(n_rows, dim)(
        wb, wrows, user_table.T, item_table.T)
    out = _make_sc_call(batch)(
        user_ids.astype(jnp.int32), item_ids.astype(jnp.int32), pu, pi)
    return out.reshape(batch, 1)
